# baseline (speedup 1.0000x reference)
import jax
import jax.numpy as jnp
from jax import lax
from jax.experimental import pallas as pl
from jax.experimental.pallas import tpu as pltpu
from jax.experimental.pallas import tpu_sc as plsc

B, L, D = 4, 8192, 768
NW = 32
SEQ_PER_W = L // NW
R = 32
N_TILES = SEQ_PER_W // R
TILE = R * D
VECS = TILE // 16


def _body(x_hbm, tab_hbm, out_hbm, t_buf, x_buf):
    c = lax.axis_index("c")
    s = lax.axis_index("s")
    wid = s * 2 + c
    seq0 = wid * SEQ_PER_W

    def add_step(i, carry):
        sl = pl.ds(i * 16, 16)
        plsc.addupdate(x_buf.at[sl], t_buf[sl])
        return carry

    def tile_step(t, carry):
        row0 = seq0 + t * R
        pltpu.sync_copy(tab_hbm.at[pl.ds(row0 * D, TILE)], t_buf)
        for b in range(B):
            src0 = (b * L + row0) * D
            pltpu.sync_copy(x_hbm.at[pl.ds(src0, TILE)], x_buf)
            lax.fori_loop(0, VECS, add_step, None, unroll=8)
            pltpu.sync_copy(x_buf, out_hbm.at[pl.ds(src0, TILE)])
        return carry

    lax.fori_loop(0, N_TILES, tile_step, None)


@jax.jit
def kernel(X, table):
    k = pl.kernel(
        _body,
        out_type=jax.ShapeDtypeStruct((B * L * D,), jnp.float32),
        mesh=plsc.VectorSubcoreMesh(core_axis_name="c", subcore_axis_name="s"),
        scratch_types=[
            pltpu.VMEM((TILE,), jnp.float32),
            pltpu.VMEM((TILE,), jnp.float32),
        ],
    )
    out = k(X.reshape(-1), table.reshape(-1))
    return out.reshape(B, L, D)
